# baseline (speedup 1.0000x reference)
import jax
import jax.numpy as jnp
from jax import lax
from jax.experimental import pallas as pl
from jax.experimental.pallas import tpu as pltpu
from jax.experimental.pallas import tpu_sc as plsc

BUFFER_SIZE = 65536
VALUE_DIM = 256
BATCH = 16384

_INFO = plsc.get_sparse_core_info()
NUM_CORES = _INFO.num_cores
NUM_SUBCORES = _INFO.num_subcores
LANES = _INFO.num_lanes
SCRANGE = BUFFER_SIZE // NUM_CORES
RBITS = 15
STRIPE = SCRANGE // NUM_SUBCORES
SLICE = BATCH // NUM_SUBCORES
SVECS = SLICE // LANES
PUBCAP = SLICE + 128
UNROLL = 4
CH = 16
CAPQ = BATCH + LANES
CAPF = BATCH + 2 * CH
NBUF = 8
PF = 6


def _body(mem_hbm, idx_hbm, val_hbm, off_hbm, out_hbm,
          stable, idx_sl, off_sl, myidx, myoff, cstage, cbuf,
          lists_all, qenc2, list_i, list_x, pubs, counts_sh,
          r0, r1, r2, r3, r4, r5, r6, r7,
          g0, g1, g2, g3, g4, g5, g6, g7,
          s0, s1, s2, s3, s4, s5, s6, s7,
          sem_i, sem_o, sem_m):
  rows = [r0, r1, r2, r3, r4, r5, r6, r7]
  semg = [g0, g1, g2, g3, g4, g5, g6, g7]
  sems = [s0, s1, s2, s3, s4, s5, s6, s7]
  cid = lax.axis_index("c")
  sid = lax.axis_index("s")
  lo = cid * SCRANGE
  slo = sid * STRIPE
  base = sid * SLICE
  iota = lax.iota(jnp.int32, LANES)
  usc = jnp.uint32(SCRANGE)
  ust = jnp.uint32(STRIPE)

  cp_i = pltpu.async_copy(idx_hbm.at[pl.ds(base, SLICE)], idx_sl, sem_i)
  cp_o = pltpu.async_copy(off_hbm.at[pl.ds(base, SLICE)], off_sl, sem_o)

  neg1 = jnp.full((LANES,), -1, jnp.int32)

  def init_body(k, _):
    for u in range(4):
      stable[pl.ds((k * 4 + u) * LANES, LANES)] = neg1
    return 0

  lax.fori_loop(0, STRIPE // LANES // 4, init_body, 0)
  cp_i.wait()
  cp_o.wait()

  def scan_body(k0, carry):
    pi, po = carry
    ivs, ovs = [], []
    for u in range(UNROLL):
      k = k0 * UNROLL + u
      ivs.append(idx_sl[pl.ds(k * LANES, LANES)])
      ovs.append(off_sl[pl.ds(k * LANES, LANES)])
    for u in range(UNROLL):
      k = k0 * UNROLL + u
      iv, ov = ivs[u], ovs[u]
      jv = iota + (base + k * LANES)
      li = iv - lo
      m = plsc.bitcast(li, jnp.uint32) < usc
      m32 = m.astype(jnp.int32)
      qi = pi + jnp.cumsum(m32) - m32
      plsc.store_scatter(myidx, [qi], (jv << RBITS) | li, mask=m)
      pi = pi + plsc.all_reduce_population_count(m)

      lo_ = ov - lo
      mo = plsc.bitcast(lo_, jnp.uint32) < usc
      mo32 = mo.astype(jnp.int32)
      qo = po + jnp.cumsum(mo32) - mo32
      plsc.store_scatter(myoff, [qo], (jv << RBITS) | lo_, mask=mo)
      po = po + plsc.all_reduce_population_count(mo)
    return pi, po

  zero = jnp.zeros((LANES,), jnp.int32)
  pi, po = lax.fori_loop(0, SVECS // UNROLL, scan_body, (zero, zero))
  ni = jnp.max(pi)
  no = jnp.max(po)

  def padpub(buf, n):
    @pl.when(n % LANES != 0)
    def _():
      lastq = jnp.full((LANES,), n - 1, jnp.int32)
      le = plsc.load_gather(buf, [lastq])
      plsc.store_scatter(buf, [n + iota], le)

  padpub(myidx, ni)
  padpub(myoff, no)

  cstage[pl.ds(0, LANES)] = jnp.broadcast_to(ni | (no << 16), (LANES,))
  pltpu.sync_copy(myidx, pubs.at[sid * 2])
  pltpu.sync_copy(myoff, pubs.at[sid * 2 + 1])
  pltpu.sync_copy(cstage, counts_sh.at[sid])
  plsc.subcore_barrier()

  pltpu.sync_copy(counts_sh, cbuf)

  def fetch(chan):
    for r in range(NUM_SUBCORES):
      pltpu.async_copy(pubs.at[r * 2 + chan],
                       lists_all.at[pl.ds(r * PUBCAP, PUBCAP)], sem_m)
    for r in range(NUM_SUBCORES):
      pltpu.make_async_copy(pubs.at[r * 2 + chan],
                            lists_all.at[pl.ds(r * PUBCAP, PUBCAP)],
                            sem_m).wait()

  def counts(r):
    cv = cbuf[r, pl.ds(0, LANES)]
    return jnp.max(cv & 0xFFFF), jnp.max(cv >> 16)

  fetch(0)
  false16 = jnp.zeros((LANES,), jnp.bool_)

  def build_list(r, acc):
    ni_r, _ = counts(r)

    def body(k, a):
      enc = lists_all[pl.ds(r * PUBCAP + k * LANES, LANES)]
      jv = enc >> RBITS
      ls = (enc & (SCRANGE - 1)) - slo
      m = plsc.bitcast(ls, jnp.uint32) < ust
      plsc.store_scatter(stable, [ls], jv, mask=m)
      w = plsc.load_gather(stable, [ls], mask=m)
      return a | (m & (w < jv))

    return lax.fori_loop(0, (ni_r + LANES - 1) // LANES, body, acc)

  lost = false16
  for r in range(NUM_SUBCORES):
    lost = build_list(r, lost)

  def fix_pass(anyw):
    acc = false16

    def fix_list(r, a0):
      ni_r, _ = counts(r)

      def body(k, a):
        enc = lists_all[pl.ds(r * PUBCAP + k * LANES, LANES)]
        jv = enc >> RBITS
        ls = (enc & (SCRANGE - 1)) - slo
        m = plsc.bitcast(ls, jnp.uint32) < ust
        w = plsc.load_gather(stable, [ls], mask=m)
        wrong = m & (w < jv)
        plsc.store_scatter(stable, [ls], jv, mask=wrong)
        return a | wrong

      return lax.fori_loop(0, (ni_r + LANES - 1) // LANES, body, a0)

    for r in range(NUM_SUBCORES):
      acc = fix_list(r, acc)
    return jnp.any(acc)

  lax.while_loop(lambda s: s, fix_pass, jnp.any(lost))

  fetch(1)

  def filt_list(r, pq0):
    _, no_r = counts(r)

    def body(k, q0):
      enc = lists_all[pl.ds(r * PUBCAP + k * LANES, LANES)]
      ls = (enc & (SCRANGE - 1)) - slo
      m = plsc.bitcast(ls, jnp.uint32) < ust
      m32 = m.astype(jnp.int32)
      q = q0 + jnp.cumsum(m32) - m32
      plsc.store_scatter(qenc2, [q], enc, mask=m)
      return q0 + plsc.all_reduce_population_count(m)

    return lax.fori_loop(0, (no_r + LANES - 1) // LANES, body, pq0)

  pq = zero
  for r in range(NUM_SUBCORES):
    pq = filt_list(r, pq)
  n_in = jnp.max(pq)

  @pl.when(n_in % LANES != 0)
  def _():
    lastq = jnp.full((LANES,), n_in - 1, jnp.int32)
    le = plsc.load_gather(qenc2, [lastq])
    plsc.store_scatter(qenc2, [n_in + iota], le)

  n_inr = ((n_in + LANES - 1) // LANES) * LANES

  def split_body(k, carry):
    ph, pm = carry
    enc = qenc2[pl.ds(k * LANES, LANES)]
    pos = enc >> RBITS
    li = enc & (SCRANGE - 1)
    r = plsc.load_gather(stable, [li - slo])
    hit = r >= 0
    miss = ~hit
    h32 = hit.astype(jnp.int32)
    m32 = miss.astype(jnp.int32)
    hq = ph + jnp.cumsum(h32) - h32
    mq = (CAPF - 1) - (pm + jnp.cumsum(m32) - m32)
    plsc.store_scatter(list_i, [hq], pos, mask=hit)
    plsc.store_scatter(list_x, [hq], r, mask=hit)
    plsc.store_scatter(list_i, [mq], pos, mask=miss)
    plsc.store_scatter(list_x, [mq], li + lo, mask=miss)
    ph = ph + plsc.all_reduce_population_count(hit)
    pm = pm + plsc.all_reduce_population_count(miss)
    return ph, pm

  ph, pm = lax.fori_loop(0, n_inr // LANES, split_body, (zero, zero))
  n_hit = jnp.max(ph)
  n_miss = jnp.max(pm)

  def pad(n, flat_of):
    @pl.when(n % CH != 0)
    def _():
      lastq = flat_of(jnp.full((LANES,), n - 1, jnp.int32))
      li_ = plsc.load_gather(list_i, [lastq])
      lx_ = plsc.load_gather(list_x, [lastq])
      for u in range(CH // LANES):
        tail = flat_of(n + u * LANES + iota)
        plsc.store_scatter(list_i, [tail], li_)
        plsc.store_scatter(list_x, [tail], lx_)

  pad(n_hit, lambda t: t)
  pad(n_miss, lambda t: (CAPF - 1) - t)

  def move(src_hbm, n, start_of):
    nch = (n + CH - 1) // CH

    def xs(c):
      return list_x.at[pl.ds(start_of(c), CH)]

    def js(c):
      return list_i.at[pl.ds(start_of(c), CH)]

    for b in range(PF):
      @pl.when(b < nch)
      def _(b=b):
        pltpu.async_copy(src_hbm.at[xs(b)], rows[b], semg[b])

    def outer(t, _):
      c0 = t * NBUF
      for b in range(NBUF):
        c = c0 + b

        @pl.when(c < nch)
        def _(b=b, c=c):
          pltpu.make_async_copy(src_hbm.at[xs(c)], rows[b], semg[b]).wait()
          pltpu.async_copy(rows[b], out_hbm.at[js(c)], sems[b])
          @pl.when(c + PF < nch)
          def _():
            b2 = (b + PF) % NBUF

            @pl.when(c >= NBUF - PF)
            def _():
              pltpu.make_async_copy(rows[b2], out_hbm.at[js(0)],
                                    sems[b2]).wait()
            pltpu.async_copy(src_hbm.at[xs(c + PF)], rows[b2], semg[b2])
      return 0

    lax.fori_loop(0, (nch + NBUF - 1) // NBUF, outer, 0)

    for b in range(NBUF):
      @pl.when(b < nch)
      def _(b=b):
        pltpu.make_async_copy(rows[b], out_hbm.at[js(0)], sems[b]).wait()

  move(val_hbm, n_hit, lambda c: c * CH)
  move(mem_hbm, n_miss, lambda c: CAPF - (c + 1) * CH)


@jax.jit
def kernel(mem, idx, val, offset):
  mesh = plsc.VectorSubcoreMesh(core_axis_name="c", subcore_axis_name="s")
  fn = pl.kernel(
      _body,
      out_type=jax.ShapeDtypeStruct((BATCH, VALUE_DIM), jnp.float32),
      mesh=mesh,
      scratch_types=(
          [
              pltpu.VMEM((STRIPE,), jnp.int32),
              pltpu.VMEM((SLICE,), jnp.int32),
              pltpu.VMEM((SLICE,), jnp.int32),
              pltpu.VMEM((PUBCAP,), jnp.int32),
              pltpu.VMEM((PUBCAP,), jnp.int32),
              pltpu.VMEM((LANES,), jnp.int32),
              pltpu.VMEM((NUM_SUBCORES, LANES), jnp.int32),
              pltpu.VMEM((NUM_SUBCORES * PUBCAP,), jnp.int32),
              pltpu.VMEM((CAPQ,), jnp.int32),
              pltpu.VMEM((CAPF,), jnp.int32),
              pltpu.VMEM((CAPF,), jnp.int32),
              pltpu.VMEM_SHARED((NUM_SUBCORES * 2, PUBCAP), jnp.int32),
              pltpu.VMEM_SHARED((NUM_SUBCORES, LANES), jnp.int32),
          ]
          + [pltpu.VMEM((CH, VALUE_DIM), jnp.float32) for _ in range(NBUF)]
          + [pltpu.SemaphoreType.DMA for _ in range(2 * NBUF + 3)]
      ),
      compiler_params=pltpu.CompilerParams(needs_layout_passes=False),
  )
  return fn(mem, idx.astype(jnp.int32), val, offset.astype(jnp.int32))


# speedup vs baseline: 1.2461x; 1.2461x over previous
import jax
import jax.numpy as jnp
from jax import lax
from jax.experimental import pallas as pl
from jax.experimental.pallas import tpu as pltpu
from jax.experimental.pallas import tpu_sc as plsc

BUFFER_SIZE = 65536
VALUE_DIM = 256
BATCH = 16384

_INFO = plsc.get_sparse_core_info()
NUM_CORES = _INFO.num_cores
NUM_SUBCORES = _INFO.num_subcores
NUM_TILES = NUM_CORES * NUM_SUBCORES
LANES = _INFO.num_lanes
RANGE = BUFFER_SIZE // NUM_TILES
RBITS = 11
NVECS = BATCH // LANES
UNROLL = 8
CH = 16
CAPQ = BATCH + LANES
CAPF = BATCH + 2 * CH
NBUF = 10
PF = 8


def _body(mem_hbm, idx_hbm, val_hbm, off_hbm, out_hbm,
          table, idx_buf, off_buf, qenc, list_i, list_x,
          r0, r1, r2, r3, r4, r5, r6, r7, r8, r9,
          g0, g1, g2, g3, g4, g5, g6, g7, g8, g9,
          s0, s1, s2, s3, s4, s5, s6, s7, s8, s9,
          sem_i, sem_o):
  rows = [r0, r1, r2, r3, r4, r5, r6, r7, r8, r9]
  semg = [g0, g1, g2, g3, g4, g5, g6, g7, g8, g9]
  sems = [s0, s1, s2, s3, s4, s5, s6, s7, s8, s9]
  wid = lax.axis_index("s") * NUM_CORES + lax.axis_index("c")
  lo = wid * RANGE
  hi = lo + RANGE
  iota = lax.iota(jnp.int32, LANES)

  cp_i = pltpu.async_copy(idx_hbm, idx_buf, sem_i)
  cp_o = pltpu.async_copy(off_hbm, off_buf, sem_o)

  neg1 = jnp.full((LANES,), -1, jnp.int32)

  def init_body(k, _):
    for u in range(4):
      table[pl.ds((k * 4 + u) * LANES, LANES)] = neg1
    return 0

  lax.fori_loop(0, RANGE // LANES // 4, init_body, 0)
  cp_i.wait()
  cp_o.wait()

  def scan_body(k0, carry):
    acc, pq = carry
    ivs, ovs = [], []
    for u in range(UNROLL):
      k = k0 * UNROLL + u
      ivs.append(idx_buf[pl.ds(k * LANES, LANES)])
      ovs.append(off_buf[pl.ds(k * LANES, LANES)])
    lis, jvs, ms = [], [], []
    for u in range(UNROLL):
      k = k0 * UNROLL + u
      iv, ov = ivs[u], ovs[u]
      jv = iota + k * LANES
      li = iv - lo
      m = plsc.bitcast(li, jnp.uint32) < jnp.uint32(RANGE)
      plsc.store_scatter(table, [li], jv, mask=m)
      lis.append(li); jvs.append(jv); ms.append(m)

      lo_ = ov - lo
      mo = plsc.bitcast(lo_, jnp.uint32) < jnp.uint32(RANGE)
      mo32 = mo.astype(jnp.int32)
      q = pq + jnp.cumsum(mo32) - mo32
      enc = (jv << RBITS) | lo_
      plsc.store_scatter(qenc, [q], enc, mask=mo)
      pq = pq + plsc.all_reduce_population_count(mo)
    for u in range(UNROLL):
      w = plsc.load_gather(table, [lis[u]], mask=ms[u])
      acc = acc | (ms[u] & (w < jvs[u]))
    return acc, pq

  false16 = jnp.zeros((LANES,), jnp.bool_)
  zero = jnp.zeros((LANES,), jnp.int32)
  lost, pq = lax.fori_loop(0, NVECS // UNROLL, scan_body, (false16, zero))
  n_in = jnp.max(pq)

  def fix_pass(anyw):
    def body(k, acc):
      iv = idx_buf[pl.ds(k * LANES, LANES)]
      jv = iota + k * LANES
      m = (iv >= lo) & (iv < hi)
      li = iv - lo
      w = plsc.load_gather(table, [li], mask=m)
      wrong = m & (w < jv)
      plsc.store_scatter(table, [li], jv, mask=wrong)
      return acc | wrong

    acc = lax.fori_loop(0, NVECS, body, false16)
    return jnp.any(acc)

  lax.while_loop(lambda s: s, fix_pass, jnp.any(lost))

  @pl.when(n_in % LANES != 0)
  def _():
    lastq = jnp.full((LANES,), n_in - 1, jnp.int32)
    le = plsc.load_gather(qenc, [lastq])
    plsc.store_scatter(qenc, [n_in + iota], le)

  n_inr = ((n_in + LANES - 1) // LANES) * LANES

  def split_body(k, carry):
    ph, pm = carry
    enc = qenc[pl.ds(k * LANES, LANES)]
    pos = enc >> RBITS
    li = enc & (RANGE - 1)
    r = plsc.load_gather(table, [li])
    hit = r >= 0
    miss = ~hit
    h32 = hit.astype(jnp.int32)
    m32 = miss.astype(jnp.int32)
    hq = ph + jnp.cumsum(h32) - h32
    mq = (CAPF - 1) - (pm + jnp.cumsum(m32) - m32)
    plsc.store_scatter(list_i, [hq], pos, mask=hit)
    plsc.store_scatter(list_x, [hq], r, mask=hit)
    plsc.store_scatter(list_i, [mq], pos, mask=miss)
    plsc.store_scatter(list_x, [mq], li + lo, mask=miss)
    ph = ph + plsc.all_reduce_population_count(hit)
    pm = pm + plsc.all_reduce_population_count(miss)
    return ph, pm

  ph, pm = lax.fori_loop(0, n_inr // LANES, split_body, (zero, zero))
  n_hit = jnp.max(ph)
  n_miss = jnp.max(pm)

  def pad(n, flat_of):
    @pl.when(n % CH != 0)
    def _():
      lastq = flat_of(jnp.full((LANES,), n - 1, jnp.int32))
      li_ = plsc.load_gather(list_i, [lastq])
      lx_ = plsc.load_gather(list_x, [lastq])
      for u in range(CH // LANES):
        tail = flat_of(n + u * LANES + iota)
        plsc.store_scatter(list_i, [tail], li_)
        plsc.store_scatter(list_x, [tail], lx_)

  pad(n_hit, lambda t: t)
  pad(n_miss, lambda t: (CAPF - 1) - t)

  def move(src_hbm, n, start_of):
    nch = (n + CH - 1) // CH

    def xs(c):
      return list_x.at[pl.ds(start_of(c), CH)]

    def js(c):
      return list_i.at[pl.ds(start_of(c), CH)]

    for b in range(PF):
      @pl.when(b < nch)
      def _(b=b):
        pltpu.async_copy(src_hbm.at[xs(b)], rows[b], semg[b])

    def outer(t, _):
      c0 = t * NBUF
      for b in range(NBUF):
        c = c0 + b

        @pl.when(c < nch)
        def _(b=b, c=c):
          pltpu.make_async_copy(src_hbm.at[xs(c)], rows[b], semg[b]).wait()
          pltpu.async_copy(rows[b], out_hbm.at[js(c)], sems[b])
          @pl.when(c + PF < nch)
          def _():
            b2 = (b + PF) % NBUF

            @pl.when(c >= NBUF - PF)
            def _():
              pltpu.make_async_copy(rows[b2], out_hbm.at[js(0)],
                                    sems[b2]).wait()
            pltpu.async_copy(src_hbm.at[xs(c + PF)], rows[b2], semg[b2])
      return 0

    lax.fori_loop(0, (nch + NBUF - 1) // NBUF, outer, 0)

    for b in range(NBUF):
      @pl.when(b < nch)
      def _(b=b):
        pltpu.make_async_copy(rows[b], out_hbm.at[js(0)], sems[b]).wait()

  move(val_hbm, n_hit, lambda c: c * CH)
  move(mem_hbm, n_miss, lambda c: CAPF - (c + 1) * CH)


@jax.jit
def kernel(mem, idx, val, offset):
  mesh = plsc.VectorSubcoreMesh(core_axis_name="c", subcore_axis_name="s")
  fn = pl.kernel(
      _body,
      out_type=jax.ShapeDtypeStruct((BATCH, VALUE_DIM), jnp.float32),
      mesh=mesh,
      scratch_types=(
          [
              pltpu.VMEM((RANGE,), jnp.int32),
              pltpu.VMEM((BATCH,), jnp.int32),
              pltpu.VMEM((BATCH,), jnp.int32),
              pltpu.VMEM((CAPQ,), jnp.int32),
              pltpu.VMEM((CAPF,), jnp.int32),
              pltpu.VMEM((CAPF,), jnp.int32),
          ]
          + [pltpu.VMEM((CH, VALUE_DIM), jnp.float32) for _ in range(NBUF)]
          + [pltpu.SemaphoreType.DMA for _ in range(2 * NBUF + 2)]
      ),
      compiler_params=pltpu.CompilerParams(needs_layout_passes=False),
  )
  return fn(mem, idx.astype(jnp.int32), val, offset.astype(jnp.int32))
